# baseline (speedup 1.0000x reference)
import math

import jax
import jax.numpy as jnp
from jax.experimental import pallas as pl
from jax.experimental.pallas import tpu as pltpu

D_MODEL_ = 256
POS_DIM_ = 32
_NT = 8


def _fourier_table(M, dtype):
    half = POS_DIM_ // 2
    freqs = jnp.exp(jnp.linspace(math.log(1.0), math.log(1000.0), half))
    idx = jnp.arange(M, dtype=dtype)
    pos = jnp.clip(idx / max(1, M - 1), 0.0, 1.0)
    ang = pos[:, None] * freqs
    out = jnp.concatenate([jnp.sin(ang), jnp.cos(ang)], axis=-1)
    if out.shape[-1] < POS_DIM_:
        out = jnp.concatenate(
            [out, jnp.zeros(out.shape[:-1] + (POS_DIM_ - out.shape[-1],), out.dtype)],
            axis=-1)
    return out.astype(dtype)


def _pool_kernel(bounds_ref, x_ref, w_ref, bias_ref, ff_ref, o_ref, acc_ref,
                 *, T):
    Tc = x_ref.shape[1]
    M = bounds_ref.shape[1]
    tc = pl.program_id(1)

    s = bounds_ref[0, :, 0]
    e = bounds_ref[0, :, 1]
    s = jnp.clip(s, 0, T - 1)
    e = jnp.minimum(e, T)
    e = jnp.maximum(s + 1, e)

    t16 = jax.lax.broadcasted_iota(jnp.int16, (M, Tc), 1)
    base = (tc * Tc).astype(jnp.int16)
    s16 = s.astype(jnp.int16) - base
    e16 = e.astype(jnp.int16) - base
    mask = (t16 >= s16[:, None]) & (t16 < e16[:, None])
    maskf = jnp.where(mask, jnp.bfloat16(1.0), jnp.bfloat16(0.0))

    part = jnp.dot(maskf, x_ref[0].astype(jnp.bfloat16),
                   preferred_element_type=jnp.float32)

    @pl.when(tc == 0)
    def _init():
        acc_ref[...] = part

    @pl.when(tc != 0)
    def _accum():
        acc_ref[...] += part

    @pl.when(tc == _NT - 1)
    def _finish():
        inv = 1.0 / (e - s).astype(jnp.float32)
        mean = acc_ref[...] * inv[:, None]
        w_top = w_ref[:D_MODEL_, :]
        w_bot = w_ref[D_MODEL_:, :]
        out = jnp.dot(mean, w_top, preferred_element_type=jnp.float32)
        out += jnp.dot(ff_ref[...], w_bot, preferred_element_type=jnp.float32)
        out += bias_ref[...][None, :]
        o_ref[0] = out


def kernel(frame_emb, beat_bounds, W, b):
    B, T, D = frame_emb.shape
    M = beat_bounds.shape[1]
    Tc = T // _NT
    bounds = beat_bounds.astype(jnp.int32)
    ff = _fourier_table(M, frame_emb.dtype)

    import functools
    return pl.pallas_call(
        functools.partial(_pool_kernel, T=T),
        grid=(B, _NT),
        in_specs=[
            pl.BlockSpec((1, M, 2), lambda i, j: (i, 0, 0)),
            pl.BlockSpec((1, Tc, D), lambda i, j: (i, j, 0)),
            pl.BlockSpec((D + POS_DIM_, D), lambda i, j: (0, 0)),
            pl.BlockSpec((D,), lambda i, j: (0,)),
            pl.BlockSpec((M, POS_DIM_), lambda i, j: (0, 0)),
        ],
        out_specs=pl.BlockSpec((1, M, D), lambda i, j: (i, 0, 0)),
        out_shape=jax.ShapeDtypeStruct((B, M, D), frame_emb.dtype),
        scratch_shapes=[pltpu.VMEM((M, D), jnp.float32)],
        compiler_params=pltpu.CompilerParams(
            dimension_semantics=("arbitrary", "arbitrary")),
    )(bounds, frame_emb, W, b, ff)


# speedup vs baseline: 2.1158x; 2.1158x over previous
import math

import jax
import jax.numpy as jnp
from jax.experimental import pallas as pl
from jax.experimental.pallas import tpu as pltpu

D_MODEL_ = 256
POS_DIM_ = 32
_NT = 2


def _fourier_table(M, dtype):
    half = POS_DIM_ // 2
    freqs = jnp.exp(jnp.linspace(math.log(1.0), math.log(1000.0), half))
    idx = jnp.arange(M, dtype=dtype)
    pos = jnp.clip(idx / max(1, M - 1), 0.0, 1.0)
    ang = pos[:, None] * freqs
    out = jnp.concatenate([jnp.sin(ang), jnp.cos(ang)], axis=-1)
    if out.shape[-1] < POS_DIM_:
        out = jnp.concatenate(
            [out, jnp.zeros(out.shape[:-1] + (POS_DIM_ - out.shape[-1],), out.dtype)],
            axis=-1)
    return out.astype(dtype)


def _pool_kernel(bounds_ref, x_ref, w_ref, bias_ref, ff_ref, o_ref, acc_ref,
                 *, T):
    Tc = x_ref.shape[1]
    M = bounds_ref.shape[1]
    tc = pl.program_id(1)

    s = bounds_ref[0, :, 0]
    e = bounds_ref[0, :, 1]
    s = jnp.clip(s, 0, T - 1)
    e = jnp.minimum(e, T)
    e = jnp.maximum(s + 1, e)

    t16 = jax.lax.broadcasted_iota(jnp.int16, (M, Tc), 1)
    base = (tc * Tc).astype(jnp.int16)
    s16 = s.astype(jnp.int16) - base
    e16 = e.astype(jnp.int16) - base
    mask = (t16 >= s16[:, None]) & (t16 < e16[:, None])
    maskf = jnp.where(mask, jnp.bfloat16(1.0), jnp.bfloat16(0.0))

    part = jnp.dot(maskf, x_ref[0].astype(jnp.bfloat16),
                   preferred_element_type=jnp.float32)

    @pl.when(tc == 0)
    def _init():
        acc_ref[...] = part

    @pl.when(tc != 0)
    def _accum():
        acc_ref[...] += part

    @pl.when(tc == _NT - 1)
    def _finish():
        inv = 1.0 / (e - s).astype(jnp.float32)
        mean = acc_ref[...] * inv[:, None]
        w_top = w_ref[:D_MODEL_, :]
        w_bot = w_ref[D_MODEL_:, :]
        out = jnp.dot(mean, w_top, preferred_element_type=jnp.float32)
        out += jnp.dot(ff_ref[...], w_bot, preferred_element_type=jnp.float32)
        out += bias_ref[...][None, :]
        o_ref[0] = out


def kernel(frame_emb, beat_bounds, W, b):
    B, T, D = frame_emb.shape
    M = beat_bounds.shape[1]
    Tc = T // _NT
    bounds = beat_bounds.astype(jnp.int32)
    ff = _fourier_table(M, frame_emb.dtype)

    import functools
    return pl.pallas_call(
        functools.partial(_pool_kernel, T=T),
        grid=(B, _NT),
        in_specs=[
            pl.BlockSpec((1, M, 2), lambda i, j: (i, 0, 0)),
            pl.BlockSpec((1, Tc, D), lambda i, j: (i, j, 0)),
            pl.BlockSpec((D + POS_DIM_, D), lambda i, j: (0, 0)),
            pl.BlockSpec((D,), lambda i, j: (0,)),
            pl.BlockSpec((M, POS_DIM_), lambda i, j: (0, 0)),
        ],
        out_specs=pl.BlockSpec((1, M, D), lambda i, j: (i, 0, 0)),
        out_shape=jax.ShapeDtypeStruct((B, M, D), frame_emb.dtype),
        scratch_shapes=[pltpu.VMEM((M, D), jnp.float32)],
        compiler_params=pltpu.CompilerParams(
            dimension_semantics=("arbitrary", "arbitrary")),
    )(bounds, frame_emb, W, b, ff)
